# baseline (speedup 1.0000x reference)
import functools

import jax
import jax.numpy as jnp
from jax.experimental import pallas as pl
from jax.experimental.pallas import tpu as pltpu

_TOPK = 10
_PAD = 16
_LANES = 128
_INT_MAX = jnp.iinfo(jnp.int32).max
_NEG_INF = float("-inf")


def _retrieval_kernel(nblk, bk, q_ref, k2_ref, keys_ref, vals_ref, idx_ref,
                      q2_ref, m_ref, s_ref, topv_ref, topi_ref, y_ref):
    j = pl.program_id(0)
    qn = q_ref.shape[0]
    nseg = y_ref.shape[1] // _LANES

    @pl.when(j == 0)
    def _init():
        q = q_ref[:]
        q2_ref[:] = jnp.sum(q * q, axis=1, keepdims=True)
        m_ref[:] = jnp.full(m_ref.shape, _NEG_INF, jnp.float32)
        s_ref[:] = jnp.zeros(s_ref.shape, jnp.float32)
        topv_ref[:] = jnp.full(topv_ref.shape, _NEG_INF, jnp.float32)
        topi_ref[:] = jnp.full(topi_ref.shape, _INT_MAX, jnp.int32)
        y_ref[:] = jnp.full(y_ref.shape, _NEG_INF, jnp.float32)

    kb = keys_ref[:]
    xy = jax.lax.dot_general(q_ref[:], kb, (((1,), (1,)), ((), ())),
                             preferred_element_type=jnp.float32)
    x = 2.0 * xy - (q2_ref[:] + k2_ref[0])

    theta = topv_ref[:, _TOPK - 1:_TOPK]
    cmp = x > theta
    counts = jnp.sum(cmp, axis=1, keepdims=True)
    trip = jnp.minimum(jnp.max(counts), _TOPK)
    yf = jnp.concatenate(
        [jnp.where(cmp, x, _NEG_INF),
         jnp.full((qn, nseg * _LANES - bk), _NEG_INF, jnp.float32)], axis=1)

    lane128 = jax.lax.broadcasted_iota(jnp.int32, (qn, _LANES), 1)

    def _fold():
        fv = jnp.full((qn, _LANES), _NEG_INF, jnp.float32)
        fi = jnp.full((qn, _LANES), _INT_MAX, jnp.int32)
        gv = jnp.full((qn, _LANES), _NEG_INF, jnp.float32)
        gi = jnp.full((qn, _LANES), _INT_MAX, jnp.int32)
        cnt = jnp.zeros((qn, _LANES), jnp.int32)
        for c in range(nseg):
            seg = yf[:, c * _LANES:(c + 1) * _LANES]
            sid = j * bk + c * _LANES + lane128
            b1 = (seg > fv) | ((seg == fv) & (sid < fi))
            b2 = (seg > gv) | ((seg == gv) & (sid < gi))
            nfv = jnp.where(b1, seg, fv)
            nfi = jnp.where(b1, sid, fi)
            gv = jnp.where(b1, fv, jnp.where(b2, seg, gv))
            gi = jnp.where(b1, fi, jnp.where(b2, sid, gi))
            fv, fi = nfv, nfi
            cnt = cnt + (seg > _NEG_INF).astype(jnp.int32)
        return fv, fi, gv, gi, cnt

    fv0, fi0, gv0, gi0, cnt0 = _fold()
    multi = jnp.any(cnt0 >= 3)

    @pl.when(multi)
    def _spill():
        y_ref[:] = yf

    m_old = m_ref[:]
    m_new = jnp.maximum(m_old, jnp.max(fv0, axis=1, keepdims=True))
    s_ref[:] = s_ref[:] * jnp.exp(m_old - m_new) + jnp.sum(
        jnp.exp(x - m_new), axis=1, keepdims=True)
    m_ref[:] = m_new

    lane = jax.lax.broadcasted_iota(jnp.int32, (qn, _PAD), 1)

    def _extract(fv, fi):
        mx = jnp.max(fv, axis=1, keepdims=True)
        ii = jnp.min(jnp.where(fv == mx, fi, _INT_MAX), axis=1, keepdims=True)
        cv = topv_ref[:]
        ci = topi_ref[:]
        beat = (cv > mx) | ((cv == mx) & (ci < ii))
        pos = jnp.sum(beat.astype(jnp.int32), axis=1, keepdims=True)
        sv = pltpu.roll(cv, 1, 1)
        si = pltpu.roll(ci, 1, 1)
        topv_ref[:] = jnp.where(lane < pos, cv, jnp.where(lane == pos, mx, sv))
        topi_ref[:] = jnp.where(lane < pos, ci, jnp.where(lane == pos, ii, si))
        return ii

    @pl.when(multi)
    def _slow_path():
        def _body(t, carry):
            del t
            fv, fi = carry
            ii = _extract(fv, fi)
            nfv = jnp.full((qn, _LANES), _NEG_INF, jnp.float32)
            nfi = jnp.full((qn, _LANES), _INT_MAX, jnp.int32)
            for c in range(nseg):
                sl = slice(c * _LANES, (c + 1) * _LANES)
                sid = j * bk + c * _LANES + lane128
                seg = jnp.where(sid == ii, _NEG_INF, y_ref[:, sl])
                y_ref[:, sl] = seg
                better = (seg > nfv) | ((seg == nfv) & (sid < nfi))
                nfv = jnp.where(better, seg, nfv)
                nfi = jnp.where(better, sid, nfi)
            return nfv, nfi

        jax.lax.fori_loop(0, trip, _body, (fv0, fi0))

    @pl.when(jnp.logical_not(multi))
    def _fast_path():
        def _body(t, carry):
            del t
            fv, fi, gv, gi = carry
            ii = _extract(fv, fi)
            kill = fi == ii
            return (jnp.where(kill, gv, fv),
                    jnp.where(kill, gi, fi),
                    jnp.where(kill, _NEG_INF, gv),
                    jnp.where(kill, _INT_MAX, gi))

        jax.lax.fori_loop(0, trip, _body, (fv0, fi0, gv0, gi0))

    @pl.when(j == nblk - 1)
    def _fin():
        vals_ref[:] = (topv_ref[:] - m_ref[:]) - jnp.log(s_ref[:])
        idx_ref[:] = topi_ref[:]


def kernel(queries, keys, k):
    qn, d = queries.shape
    n = keys.shape[0]
    bk = 2000 if n % 2000 == 0 else n
    nblk = n // bk
    ypad = ((bk + _LANES - 1) // _LANES) * _LANES
    k2 = jnp.sum(keys * keys, axis=1).reshape(nblk, 1, bk)
    body = functools.partial(_retrieval_kernel, nblk, bk)
    vals, idx = pl.pallas_call(
        body,
        grid=(nblk,),
        in_specs=[
            pl.BlockSpec((qn, d), lambda j: (0, 0)),
            pl.BlockSpec((1, 1, bk), lambda j: (j, 0, 0)),
            pl.BlockSpec((bk, d), lambda j: (j, 0)),
        ],
        out_specs=[
            pl.BlockSpec((qn, _PAD), lambda j: (0, 0)),
            pl.BlockSpec((qn, _PAD), lambda j: (0, 0)),
        ],
        out_shape=[
            jax.ShapeDtypeStruct((qn, _PAD), jnp.float32),
            jax.ShapeDtypeStruct((qn, _PAD), jnp.int32),
        ],
        scratch_shapes=[
            pltpu.VMEM((qn, 1), jnp.float32),
            pltpu.VMEM((qn, 1), jnp.float32),
            pltpu.VMEM((qn, 1), jnp.float32),
            pltpu.VMEM((qn, _PAD), jnp.float32),
            pltpu.VMEM((qn, _PAD), jnp.int32),
            pltpu.VMEM((qn, ypad), jnp.float32),
        ],
    )(queries, k2, keys)
    vals = vals[:, :_TOPK]
    idx = idx[:, :_TOPK] + (jnp.asarray(k, jnp.int32) - _TOPK)
    return vals, idx


# speedup vs baseline: 1.0811x; 1.0811x over previous
import functools

import jax
import jax.numpy as jnp
from jax.experimental import pallas as pl
from jax.experimental.pallas import tpu as pltpu

_TOPK = 10
_PAD = 16
_LANES = 128
_INT_MAX = jnp.iinfo(jnp.int32).max
_NEG_INF = float("-inf")


def _retrieval_kernel(nblk, bk, q_ref, k2_ref, keys_ref, vals_ref, idx_ref,
                      q2_ref, m_ref, s_ref, topv_ref, topi_ref, y_ref):
    j = pl.program_id(0)
    qn = q_ref.shape[0]
    nseg = y_ref.shape[1] // _LANES

    @pl.when(j == 0)
    def _init():
        q = q_ref[:]
        q2_ref[:] = jnp.sum(q * q, axis=1, keepdims=True)
        m_ref[:] = jnp.full(m_ref.shape, _NEG_INF, jnp.float32)
        s_ref[:] = jnp.zeros(s_ref.shape, jnp.float32)
        topv_ref[:] = jnp.full(topv_ref.shape, _NEG_INF, jnp.float32)
        topi_ref[:] = jnp.full(topi_ref.shape, _INT_MAX, jnp.int32)
        y_ref[:] = jnp.full(y_ref.shape, _NEG_INF, jnp.float32)

    kb = keys_ref[:]
    xy = jax.lax.dot_general(q_ref[:], kb, (((1,), (1,)), ((), ())),
                             preferred_element_type=jnp.float32)
    x = 2.0 * xy - (q2_ref[:] + k2_ref[0])

    theta = topv_ref[:, _TOPK - 1:_TOPK]
    cmp = x > theta
    y_ref[:, :bk] = jnp.where(cmp, x, _NEG_INF)

    lane128 = jax.lax.broadcasted_iota(jnp.int32, (qn, _LANES), 1)

    def _fold():
        fv = jnp.full((qn, _LANES), _NEG_INF, jnp.float32)
        fi = jnp.full((qn, _LANES), _INT_MAX, jnp.int32)
        gv = jnp.full((qn, _LANES), _NEG_INF, jnp.float32)
        gi = jnp.full((qn, _LANES), _INT_MAX, jnp.int32)
        cnt = jnp.zeros((qn, _LANES), jnp.int32)
        for c in range(nseg):
            seg = y_ref[:, c * _LANES:(c + 1) * _LANES]
            sid = j * bk + c * _LANES + lane128
            b1 = (seg > fv) | ((seg == fv) & (sid < fi))
            b2 = (seg > gv) | ((seg == gv) & (sid < gi))
            nfv = jnp.where(b1, seg, fv)
            nfi = jnp.where(b1, sid, fi)
            gv = jnp.where(b1, fv, jnp.where(b2, seg, gv))
            gi = jnp.where(b1, fi, jnp.where(b2, sid, gi))
            fv, fi = nfv, nfi
            cnt = cnt + (seg > _NEG_INF).astype(jnp.int32)
        return fv, fi, gv, gi, cnt

    fv0, fi0, gv0, gi0, cnt0 = _fold()
    multi = jnp.any(cnt0 >= 3)
    counts = jnp.sum(cnt0, axis=1, keepdims=True)
    trip = jnp.minimum(jnp.max(counts), _TOPK)

    m_old = m_ref[:]
    m_new = jnp.maximum(m_old, jnp.max(fv0, axis=1, keepdims=True))
    s_ref[:] = s_ref[:] * jnp.exp(m_old - m_new) + jnp.sum(
        jnp.exp(x - m_new), axis=1, keepdims=True)
    m_ref[:] = m_new

    lane = jax.lax.broadcasted_iota(jnp.int32, (qn, _PAD), 1)

    def _extract(fv, fi):
        mx = jnp.max(fv, axis=1, keepdims=True)
        ii = jnp.min(jnp.where(fv == mx, fi, _INT_MAX), axis=1, keepdims=True)
        cv = topv_ref[:]
        ci = topi_ref[:]
        beat = (cv > mx) | ((cv == mx) & (ci < ii))
        pos = jnp.sum(beat.astype(jnp.int32), axis=1, keepdims=True)
        sv = pltpu.roll(cv, 1, 1)
        si = pltpu.roll(ci, 1, 1)
        topv_ref[:] = jnp.where(lane < pos, cv, jnp.where(lane == pos, mx, sv))
        topi_ref[:] = jnp.where(lane < pos, ci, jnp.where(lane == pos, ii, si))
        return ii

    @pl.when(multi)
    def _slow_path():
        def _body(t, carry):
            del t
            fv, fi = carry
            ii = _extract(fv, fi)
            nfv = jnp.full((qn, _LANES), _NEG_INF, jnp.float32)
            nfi = jnp.full((qn, _LANES), _INT_MAX, jnp.int32)
            for c in range(nseg):
                sl = slice(c * _LANES, (c + 1) * _LANES)
                sid = j * bk + c * _LANES + lane128
                seg = jnp.where(sid == ii, _NEG_INF, y_ref[:, sl])
                y_ref[:, sl] = seg
                better = (seg > nfv) | ((seg == nfv) & (sid < nfi))
                nfv = jnp.where(better, seg, nfv)
                nfi = jnp.where(better, sid, nfi)
            return nfv, nfi

        jax.lax.fori_loop(0, trip, _body, (fv0, fi0))

    @pl.when(jnp.logical_not(multi))
    def _fast_path():
        def _body(t, carry):
            del t
            fv, fi, gv, gi = carry
            ii = _extract(fv, fi)
            kill = fi == ii
            return (jnp.where(kill, gv, fv),
                    jnp.where(kill, gi, fi),
                    jnp.where(kill, _NEG_INF, gv),
                    jnp.where(kill, _INT_MAX, gi))

        jax.lax.fori_loop(0, trip, _body, (fv0, fi0, gv0, gi0))

    @pl.when(j == nblk - 1)
    def _fin():
        vals_ref[:] = (topv_ref[:] - m_ref[:]) - jnp.log(s_ref[:])
        idx_ref[:] = topi_ref[:]


def kernel(queries, keys, k):
    qn, d = queries.shape
    n = keys.shape[0]
    bk = 2000 if n % 2000 == 0 else n
    nblk = n // bk
    ypad = ((bk + _LANES - 1) // _LANES) * _LANES
    k2 = jnp.sum(keys * keys, axis=1).reshape(nblk, 1, bk)
    body = functools.partial(_retrieval_kernel, nblk, bk)
    vals, idx = pl.pallas_call(
        body,
        grid=(nblk,),
        in_specs=[
            pl.BlockSpec((qn, d), lambda j: (0, 0)),
            pl.BlockSpec((1, 1, bk), lambda j: (j, 0, 0)),
            pl.BlockSpec((bk, d), lambda j: (j, 0)),
        ],
        out_specs=[
            pl.BlockSpec((qn, _PAD), lambda j: (0, 0)),
            pl.BlockSpec((qn, _PAD), lambda j: (0, 0)),
        ],
        out_shape=[
            jax.ShapeDtypeStruct((qn, _PAD), jnp.float32),
            jax.ShapeDtypeStruct((qn, _PAD), jnp.int32),
        ],
        scratch_shapes=[
            pltpu.VMEM((qn, 1), jnp.float32),
            pltpu.VMEM((qn, 1), jnp.float32),
            pltpu.VMEM((qn, 1), jnp.float32),
            pltpu.VMEM((qn, _PAD), jnp.float32),
            pltpu.VMEM((qn, _PAD), jnp.int32),
            pltpu.VMEM((qn, ypad), jnp.float32),
        ],
    )(queries, k2, keys)
    vals = vals[:, :_TOPK]
    idx = idx[:, :_TOPK] + (jnp.asarray(k, jnp.int32) - _TOPK)
    return vals, idx
